# baseline (speedup 1.0000x reference)
import functools
import math

import jax
import jax.numpy as jnp
from jax import lax
from jax.experimental import pallas as pl
from jax.experimental.pallas import tpu as pltpu
from jax.experimental.pallas import tpu_sc as plsc

D_MODEL = 64
_SCALE = math.sqrt(D_MODEL)
_LANES = 128

_SPLAT_DNUMS = lax.GatherDimensionNumbers(
    offset_dims=(), collapsed_slice_dims=(0,), start_index_map=(0,)
)


def _splat(vec, k):
    idx = jnp.full((16, 1), k, jnp.int32)
    return lax.gather(
        vec, idx, _SPLAT_DNUMS, slice_sizes=(1,),
        mode=lax.GatherScatterMode.PROMISE_IN_BOUNDS,
    )


@functools.lru_cache(maxsize=None)
def _build(V, D, B):
    info = plsc.get_sparse_core_info()
    NC, NS, L = info.num_cores, info.num_subcores, info.num_lanes
    NW = NC * NS
    assert B % NW == 0 and V % 2 == 0
    b_per_w = B // NW
    C = 128
    assert b_per_w % C == 0
    n_chunks = b_per_w // C
    NBUF = 2
    mesh = plsc.VectorSubcoreMesh(core_axis_name="c", subcore_axis_name="s")

    n_tiles = (V + _LANES - 1) // _LANES
    KT = 2
    RW = KT * _LANES
    n_rounds = -(-n_tiles // KT)
    jobs_pw = -(-n_rounds // NW)
    if jobs_pw % 2:
        jobs_pw += 1
    rows_per_round = RW // 2
    V2 = n_rounds * rows_per_round
    WPAD = 3
    n_dg = D // 8

    @functools.partial(
        pl.kernel,
        mesh=mesh,
        out_type=jax.ShapeDtypeStruct((V2, _LANES), jnp.float32),
        compiler_params=pltpu.CompilerParams(
            use_tc_tiling_on_sc=True, needs_layout_passes=False
        ),
        scratch_types=[
            pltpu.VMEM((NBUF, n_dg, 8, RW), jnp.float32),
            pltpu.VMEM((NBUF, rows_per_round, _LANES + WPAD), jnp.float32),
            [pltpu.SemaphoreType.DMA] * NBUF,
            [pltpu.SemaphoreType.DMA] * NBUF,
        ],
    )
    def pack_kernel(tt_hbm, tail_hbm, out_hbm, gbuf, wbuf, rsems, wsems):
        wid = lax.axis_index("s") * NC + lax.axis_index("c")
        p0 = jnp.minimum(wid * jobs_pw, n_rounds - jobs_pw)

        def start_read(p, b):
            @pl.when(p < n_rounds - 1)
            def _():
                for dg in range(n_dg):
                    pltpu.async_copy(
                        tt_hbm.at[pl.ds(8 * dg, 8), pl.ds(p * RW, RW)],
                        gbuf.at[b, dg],
                        rsems[b],
                    )

            @pl.when(p >= n_rounds - 1)
            def _():
                for dg in range(n_dg):
                    pltpu.async_copy(
                        tail_hbm.at[pl.ds(8 * dg, 8)], gbuf.at[b, dg], rsems[b]
                    )

        def wait_read(b):
            for dg in range(n_dg):
                pltpu.make_async_copy(
                    tail_hbm.at[pl.ds(0, 8)], gbuf.at[b, 0], rsems[b]
                ).wait()

        def start_write(p, b):
            pltpu.async_copy(
                wbuf.at[b, :, pl.ds(0, _LANES)],
                out_hbm.at[pl.ds(p * rows_per_round, rows_per_round)],
                wsems[b],
            )

        def wait_write(b):
            pltpu.make_async_copy(
                wbuf.at[b, :, pl.ds(0, _LANES)],
                out_hbm.at[pl.ds(0, rows_per_round)],
                wsems[b],
            ).wait()

        iota16 = lax.iota(jnp.int32, L)
        row_vecs = [(k * L + iota16) >> 1 for k in range(RW // L)]
        par_vec = (iota16 & 1) * D

        def transpose(b):
            wb = wbuf.at[b]

            @plsc.parallel_loop(0, D, unroll=2)
            def _in_row(d):
                dg = d >> 3
                s = d & 7
                col_vec = par_vec + d
                for k in range(RW // L):
                    vals = gbuf[b, dg, s, pl.ds(k * L, L)]
                    plsc.store_scatter(wb, [row_vecs[k], col_vec], vals)

        for b in range(NBUF):
            start_read(p0 + b, b)
        for b in range(NBUF):
            wait_read(b)
            transpose(b)
            start_read(p0 + b + NBUF, b)
            start_write(p0 + b, b)

        def steady(k0, carry):
            for b in range(NBUF):
                k = k0 + b
                wait_read(b)
                wait_write(b)
                transpose(b)
                start_read(p0 + k + NBUF, b)
                start_write(p0 + k, b)
            return carry

        lax.fori_loop(1, jobs_pw // NBUF - 1, lambda g, c: steady(g * NBUF, c), 0)

        for b in range(NBUF):
            k = jobs_pw - NBUF + b
            wait_read(b)
            wait_write(b)
            transpose(b)
            start_write(p0 + k, b)
        for b in range(NBUF):
            wait_write(b)

    @functools.partial(
        pl.kernel,
        mesh=mesh,
        out_type=jax.ShapeDtypeStruct((B, D), jnp.float32),
        compiler_params=pltpu.CompilerParams(
            use_tc_tiling_on_sc=True, needs_layout_passes=False
        ),
        scratch_types=[
            pltpu.VMEM((n_chunks, C), jnp.int32),
            pltpu.VMEM((n_chunks, C), jnp.int32),
            pltpu.VMEM((NBUF, C, _LANES), jnp.float32),
            pltpu.VMEM((NBUF, C, D), jnp.float32),
            [pltpu.SemaphoreType.DMA] * NBUF,
            [pltpu.SemaphoreType.DMA] * NBUF,
        ],
    )
    def emb_kernel(
        table_hbm, xj_hbm, xp_hbm, out_hbm,
        idx_v, off_v, gbuf, wbuf, gsems, wsems,
    ):
        wid = lax.axis_index("s") * NC + lax.axis_index("c")
        base = wid * b_per_w
        pltpu.sync_copy(xj_hbm.at[wid], idx_v)
        pltpu.sync_copy(xp_hbm.at[wid], off_v)

        def start_gather(ci, b):
            pltpu.async_copy(table_hbm.at[idx_v.at[ci]], gbuf.at[b], gsems[b])

        def wait_gather(ci, b):
            pltpu.make_async_copy(
                table_hbm.at[idx_v.at[ci]], gbuf.at[b], gsems[b]
            ).wait()

        def wait_write(b):
            pltpu.make_async_copy(
                wbuf.at[b], out_hbm.at[pl.ds(base, C)], wsems[b]
            ).wait()

        def start_write(ci, b):
            pltpu.async_copy(
                wbuf.at[b], out_hbm.at[pl.ds(base + ci * C, C)], wsems[b]
            )

        def scale(ci, b):
            gb = gbuf.at[b]

            @plsc.parallel_loop(0, C // L, unroll=2)
            def _scale_group(g):
                offs = off_v[ci, pl.ds(g * L, L)]
                for rm in range(L):
                    off = _splat(offs, rm)
                    r = g * L + rm
                    row_vec = jnp.zeros((L,), jnp.int32) + r
                    for d in range(D // L):
                        col = off + (d * L + lax.iota(jnp.int32, L))
                        vals = plsc.load_gather(gb, [row_vec, col])
                        wbuf[b, r, pl.ds(d * L, L)] = vals * _SCALE

        for b in range(NBUF):
            start_gather(b, b)
        for b in range(NBUF):
            wait_gather(b, b)
            scale(b, b)
            start_gather(b + NBUF, b)
            start_write(b, b)

        def steady(g0, carry):
            for b in range(NBUF):
                ci = g0 + b
                wait_gather(ci, b)
                wait_write(b)
                scale(ci, b)
                start_gather(ci + NBUF, b)
                start_write(ci, b)
            return carry

        lax.fori_loop(1, n_chunks // NBUF - 1, lambda g, c: steady(g * NBUF, c), 0)

        for b in range(NBUF):
            ci = n_chunks - NBUF + b
            wait_gather(ci, b)
            wait_write(b)
            scale(ci, b)
            start_write(ci, b)
        for b in range(NBUF):
            wait_write(b)

    def run(table, x):
        table_t = table.T
        n_tail = V - (n_rounds - 1) * RW
        tail = jnp.pad(
            table[V - n_tail:], ((0, RW - n_tail), (0, 0))
        ).T
        packed = pack_kernel(table_t, tail)
        xj = (x >> 1).reshape(NW, n_chunks, C)
        xp = ((x & 1) << 6).reshape(NW, n_chunks, C)
        return emb_kernel(packed, xj, xp)

    return run


def kernel(x, table):
    Bdim, T = x.shape
    V, D = table.shape
    run = _build(V, D, Bdim * T)
    out = run(table, x.reshape(-1).astype(jnp.int32))
    return out.reshape(Bdim, T, D)


# speedup vs baseline: 1.0479x; 1.0479x over previous
import functools
import math

import jax
import jax.numpy as jnp
from jax import lax
from jax.experimental import pallas as pl
from jax.experimental.pallas import tpu as pltpu
from jax.experimental.pallas import tpu_sc as plsc

D_MODEL = 64
_SCALE = math.sqrt(D_MODEL)
_LANES = 128


@functools.lru_cache(maxsize=None)
def _build(V, D, B):
    info = plsc.get_sparse_core_info()
    NC, NS, L = info.num_cores, info.num_subcores, info.num_lanes
    NW = NC * NS
    assert B % NW == 0
    b_per_w = B // NW
    C = 128
    assert b_per_w % C == 0
    n_chunks = b_per_w // C
    NBUF = 4
    LOOK = 2
    assert n_chunks % NBUF == 0
    mesh = plsc.VectorSubcoreMesh(core_axis_name="c", subcore_axis_name="s")

    @functools.partial(
        pl.kernel,
        mesh=mesh,
        out_type=jax.ShapeDtypeStruct((B, _LANES), jnp.float32),
        compiler_params=pltpu.CompilerParams(
            use_tc_tiling_on_sc=True, needs_layout_passes=False
        ),
        scratch_types=[
            pltpu.VMEM((n_chunks, C), jnp.int32),
            pltpu.VMEM((NBUF, C, _LANES), jnp.float32),
            [pltpu.SemaphoreType.DMA] * NBUF,
            [pltpu.SemaphoreType.DMA] * NBUF,
        ],
    )
    def emb_kernel(table_hbm, x_hbm, out_hbm, idx_v, gbuf, gsems, wsems):
        wid = lax.axis_index("s") * NC + lax.axis_index("c")
        base = wid * b_per_w
        pltpu.sync_copy(x_hbm.at[wid], idx_v)

        def start_gather(ci, b):
            pltpu.async_copy(table_hbm.at[idx_v.at[ci]], gbuf.at[b], gsems[b])

        def wait_gather(ci, b):
            pltpu.make_async_copy(
                table_hbm.at[idx_v.at[ci]], gbuf.at[b], gsems[b]
            ).wait()

        def start_write(ci, b):
            pltpu.async_copy(
                gbuf.at[b], out_hbm.at[pl.ds(base + ci * C, C)], wsems[b]
            )

        def wait_write(b):
            pltpu.make_async_copy(
                gbuf.at[b], out_hbm.at[pl.ds(base, C)], wsems[b]
            ).wait()

        for b in range(LOOK):
            start_gather(b, b)

        for ci in range(LOOK):
            start_gather(ci + LOOK, (ci + LOOK) % NBUF)
            wait_gather(ci, ci % NBUF)
            start_write(ci, ci % NBUF)

        def steady(ci0, carry):
            for k in range(NBUF):
                ci = ci0 + k
                b_cur = (LOOK + k) % NBUF
                b_next = (2 * LOOK + k) % NBUF
                wait_write(b_next)
                start_gather(ci + LOOK, b_next)
                wait_gather(ci, b_cur)
                start_write(ci, b_cur)
            return carry

        n_steady = (n_chunks - 2 * LOOK) // NBUF
        lax.fori_loop(
            0, n_steady, lambda g, c: steady(LOOK + g * NBUF, c), 0
        )

        for k in range(LOOK):
            ci = n_chunks - LOOK + k
            wait_gather(ci, ci % NBUF)
            start_write(ci, ci % NBUF)
        for b in range(NBUF):
            wait_write(b)

    def run(table, x):
        table_p = jnp.pad(table * _SCALE, ((0, 0), (0, _LANES - D)))
        x3 = x.reshape(NW, n_chunks, C)
        return emb_kernel(table_p, x3)[:, :D]

    return run


def kernel(x, table):
    Bdim, T = x.shape
    V, D = table.shape
    run = _build(V, D, Bdim * T)
    out = run(table, x.reshape(-1).astype(jnp.int32))
    return out.reshape(Bdim, T, D)


# speedup vs baseline: 1.3781x; 1.3151x over previous
import functools
import math

import jax
import jax.numpy as jnp
from jax import lax
from jax.experimental import pallas as pl
from jax.experimental.pallas import tpu as pltpu
from jax.experimental.pallas import tpu_sc as plsc

D_MODEL = 64
_SCALE = math.sqrt(D_MODEL)
_LANES = 128


@functools.lru_cache(maxsize=None)
def _build(V, D, B):
    info = plsc.get_sparse_core_info()
    NC, NS, L = info.num_cores, info.num_subcores, info.num_lanes
    NW = NC * NS
    assert B % NW == 0
    b_per_w = B // NW
    C = 128
    assert b_per_w % C == 0
    n_chunks = b_per_w // C
    NG = 4
    NWB = 2
    LOOK = 2
    assert n_chunks % NG == 0
    mesh = plsc.VectorSubcoreMesh(core_axis_name="c", subcore_axis_name="s")

    @functools.partial(
        pl.kernel,
        mesh=mesh,
        out_type=jax.ShapeDtypeStruct((B, D), jnp.float32),
        compiler_params=pltpu.CompilerParams(use_tc_tiling_on_sc=True),
        scratch_types=[
            pltpu.VMEM((n_chunks, C), jnp.int32),
            pltpu.VMEM((NG, C, _LANES), jnp.float32),
            pltpu.VMEM((NWB, C, D), jnp.float32),
            [pltpu.SemaphoreType.DMA] * NG,
            [pltpu.SemaphoreType.DMA] * NWB,
        ],
    )
    def emb_kernel(table_hbm, x_hbm, out_hbm, idx_v, gbuf, wbuf, gsems, wsems):
        wid = lax.axis_index("s") * NC + lax.axis_index("c")
        base = wid * b_per_w
        pltpu.sync_copy(x_hbm.at[wid], idx_v)

        def start_gather(ci, b):
            pltpu.async_copy(table_hbm.at[idx_v.at[ci]], gbuf.at[b], gsems[b])

        def wait_gather(ci, b):
            pltpu.make_async_copy(
                table_hbm.at[idx_v.at[ci]], gbuf.at[b], gsems[b]
            ).wait()

        def wait_write(b):
            pltpu.make_async_copy(
                wbuf.at[b], out_hbm.at[pl.ds(base, C)], wsems[b]
            ).wait()

        def start_write(ci, b):
            pltpu.async_copy(
                wbuf.at[b], out_hbm.at[pl.ds(base + ci * C, C)], wsems[b]
            )

        def scale(gb, wb):
            @plsc.parallel_loop(0, C, unroll=8)
            def _scale_body(r):
                for d in range(D // L):
                    sl = pl.ds(d * L, L)
                    wbuf[wb, r, sl] = gbuf[gb, r, sl] * _SCALE

        for ci in range(LOOK):
            start_gather(ci, ci % NG)

        for ci in range(NWB):
            start_gather(ci + LOOK, (ci + LOOK) % NG)
            wait_gather(ci, ci % NG)
            scale(ci % NG, ci % NWB)
            start_write(ci, ci % NWB)

        def steady(ci0, carry):
            for k in range(NG):
                ci = ci0 + k
                gb = (NWB + k) % NG
                gb_next = (NWB + k + LOOK) % NG
                wb = (NWB + k) % NWB
                start_gather(ci + LOOK, gb_next)
                wait_gather(ci, gb)
                wait_write(wb)
                scale(gb, wb)
                start_write(ci, wb)
            return carry

        n_steady = (n_chunks - NWB - LOOK) // NG
        lax.fori_loop(0, n_steady, lambda g, c: steady(NWB + g * NG, c), 0)

        for k in range(LOOK):
            ci = n_chunks - LOOK + k
            wait_gather(ci, ci % NG)
            wait_write(ci % NWB)
            scale(ci % NG, ci % NWB)
            start_write(ci, ci % NWB)
        for b in range(NWB):
            wait_write(b)

    def run(table, x):
        table_p = jnp.pad(table, ((0, 0), (0, _LANES - D)))
        x3 = x.reshape(NW, n_chunks, C)
        return emb_kernel(table_p, x3)

    return run


def kernel(x, table):
    Bdim, T = x.shape
    V, D = table.shape
    run = _build(V, D, Bdim * T)
    out = run(table, x.reshape(-1).astype(jnp.int32))
    return out.reshape(Bdim, T, D)
